# Initial kernel scaffold; baseline (speedup 1.0000x reference)
import jax
import jax.numpy as jnp
from jax.experimental import pallas as pl


def kernel(inputs, embedding):
    raise NotImplementedError("write your pallas kernel here")


# speedup vs baseline: 8.3533x; 8.3533x over previous
import functools

import jax
import jax.numpy as jnp
from jax import lax
from jax.experimental import pallas as pl
from jax.experimental.pallas import tpu as pltpu
from jax.experimental.pallas import tpu_sc as plsc

N = 8192
K = 8192
D = 32
BN = 256
NB = N // BN
_LOSS_SCALE = 0.25 / (N * D)

_NW = 32
_BPW = N // _NW


def _tc_body(x_ref, e_ref, x2_ref, e2_ref,
             dist_ref, enc_ref, idx_ref, loss_ref, perp_ref,
             counts_ref, lacc_ref):
    i = pl.program_id(0)
    xt = x_ref[...]
    et = e_ref[...]
    xe = lax.dot_general(xt, et, (((1,), (1,)), ((), ())),
                         preferred_element_type=jnp.float32)
    dist = x2_ref[...] + e2_ref[...] - 2.0 * xe
    dist_ref[...] = dist

    idx = jnp.argmin(dist, axis=1).astype(jnp.int32)
    idx_ref[...] = idx[:, None]

    col = lax.broadcasted_iota(jnp.int32, (BN, K), 1)
    enc = (col == idx[:, None]).astype(jnp.float32)
    enc_ref[...] = enc

    tile_loss = jnp.sum(jnp.min(dist, axis=1))
    tile_counts = jnp.sum(enc, axis=0, keepdims=True)

    @pl.when(i == 0)
    def _():
        counts_ref[...] = tile_counts
        lacc_ref[0] = tile_loss

    @pl.when(i > 0)
    def _():
        counts_ref[...] += tile_counts
        lacc_ref[0] += tile_loss

    @pl.when(i == NB - 1)
    def _():
        loss_ref[...] = jnp.reshape(lacc_ref[0] * _LOSS_SCALE, (1, 1))
        avg = counts_ref[...] * (1.0 / N)
        perp = jnp.exp(-jnp.sum(avg * jnp.log(avg + 1e-10)))
        perp_ref[...] = jnp.reshape(perp, (1, 1))


def _tc_call(x, e, x2, e2):
    return pl.pallas_call(
        _tc_body,
        grid=(NB,),
        in_specs=[
            pl.BlockSpec((BN, D), lambda i: (i, 0)),
            pl.BlockSpec((K, D), lambda i: (0, 0)),
            pl.BlockSpec((BN, 1), lambda i: (i, 0)),
            pl.BlockSpec((1, K), lambda i: (0, 0)),
        ],
        out_specs=[
            pl.BlockSpec((BN, K), lambda i: (i, 0)),
            pl.BlockSpec((BN, K), lambda i: (i, 0)),
            pl.BlockSpec((BN, 1), lambda i: (i, 0)),
            pl.BlockSpec((1, 1), lambda i: (0, 0)),
            pl.BlockSpec((1, 1), lambda i: (0, 0)),
        ],
        out_shape=[
            jax.ShapeDtypeStruct((N, K), jnp.float32),
            jax.ShapeDtypeStruct((N, K), jnp.float32),
            jax.ShapeDtypeStruct((N, 1), jnp.int32),
            jax.ShapeDtypeStruct((1, 1), jnp.float32),
            jax.ShapeDtypeStruct((1, 1), jnp.float32),
        ],
        scratch_shapes=[
            pltpu.VMEM((1, K), jnp.float32),
            pltpu.SMEM((1,), jnp.float32),
        ],
    )(x, e, x2, e2)


def _sc_body(emb_hbm, idx_hbm, x_hbm, out_hbm, idx_v, rows_v, x_v, sem):
    c = lax.axis_index("c")
    s = lax.axis_index("s")
    wid = s * 2 + c
    base = wid * _BPW
    pltpu.sync_copy(idx_hbm.at[pl.ds(wid * 2, 2), :], idx_v)
    cps = [pltpu.async_copy(emb_hbm.at[idx_v.at[j]],
                            rows_v.at[pl.ds(j * 128, 128), :], sem)
           for j in range(2)]
    pltpu.sync_copy(x_hbm.at[pl.ds(base, _BPW), :], x_v)
    for cp in cps:
        cp.wait()

    def body(r, carry):
        for ch in range(D // 16):
            sl = pl.ds(ch * 16, 16)
            q = rows_v[r, sl]
            xv = x_v[r, sl]
            rows_v[r, sl] = xv + (q - xv)
        return carry

    lax.fori_loop(0, _BPW, body, 0)
    pltpu.sync_copy(rows_v, out_hbm.at[pl.ds(base, _BPW), :])


def _sc_call(emb_bf, idx2d, x):
    mesh = plsc.VectorSubcoreMesh(core_axis_name="c", subcore_axis_name="s")
    fn = functools.partial(
        pl.kernel,
        mesh=mesh,
        out_type=jax.ShapeDtypeStruct((N, D), jnp.float32),
        scratch_types=[
            pltpu.VMEM((2, 128), jnp.int32),
            pltpu.VMEM((_BPW, D), jnp.float32),
            pltpu.VMEM((_BPW, D), jnp.float32),
            pltpu.SemaphoreType.DMA,
        ],
        compiler_params=pltpu.CompilerParams(use_tc_tiling_on_sc=False),
    )(_sc_body)
    return fn(emb_bf, idx2d, x)


def kernel(inputs, embedding):
    x = inputs.reshape(-1, D)
    x2 = jnp.sum(x ** 2, axis=1, keepdims=True)
    e2 = jnp.sum(embedding ** 2, axis=1)[None, :]
    dist, enc, idx, loss11, perp11 = _tc_call(x, embedding, x2, e2)
    emb_bf = embedding.astype(jnp.bfloat16).astype(jnp.float32)
    qst = _sc_call(emb_bf, idx.reshape(N // 128, 128), x)
    return (loss11[0, 0], qst.reshape(inputs.shape), perp11[0, 0],
            enc, dist, idx)


# speedup vs baseline: 8.3920x; 1.0046x over previous
import functools

import jax
import jax.numpy as jnp
from jax import lax
from jax.experimental import pallas as pl
from jax.experimental.pallas import tpu as pltpu
from jax.experimental.pallas import tpu_sc as plsc

N = 8192
K = 8192
D = 32
BN = 256
NB = N // BN
_LOSS_SCALE = 0.25 / (N * D)

_NW = 32
_BPW = N // _NW


def _tc_body(x_ref, e_ref, x2_ref, e2_ref,
             dist_ref, enc_ref, idx_ref, loss_ref, perp_ref,
             counts_ref, lacc_ref):
    i = pl.program_id(0)
    xt = x_ref[...]
    et = e_ref[...]
    xe = lax.dot_general(xt, et, (((1,), (1,)), ((), ())),
                         preferred_element_type=jnp.float32)
    dist = x2_ref[...] + e2_ref[...] - 2.0 * xe
    dist_ref[...] = dist

    idx = jnp.argmin(dist, axis=1).astype(jnp.int32)
    idx_ref[...] = idx[:, None]

    col = lax.broadcasted_iota(jnp.int32, (BN, K), 1)
    enc = (col == idx[:, None]).astype(jnp.float32)
    enc_ref[...] = enc

    tile_loss = jnp.sum(jnp.min(dist, axis=1))
    tile_counts = jnp.sum(enc, axis=0, keepdims=True)

    @pl.when(i == 0)
    def _():
        counts_ref[...] = tile_counts
        lacc_ref[0] = tile_loss

    @pl.when(i > 0)
    def _():
        counts_ref[...] += tile_counts
        lacc_ref[0] += tile_loss

    @pl.when(i == NB - 1)
    def _():
        loss_ref[...] = jnp.reshape(lacc_ref[0] * _LOSS_SCALE, (1, 1))
        avg = counts_ref[...] * (1.0 / N)
        perp = jnp.exp(-jnp.sum(avg * jnp.log(avg + 1e-10)))
        perp_ref[...] = jnp.reshape(perp, (1, 1))


def _tc_call(x, e, x2, e2):
    return pl.pallas_call(
        _tc_body,
        grid=(NB,),
        in_specs=[
            pl.BlockSpec((BN, D), lambda i: (i, 0)),
            pl.BlockSpec((K, D), lambda i: (0, 0)),
            pl.BlockSpec((BN, 1), lambda i: (i, 0)),
            pl.BlockSpec((1, K), lambda i: (0, 0)),
        ],
        out_specs=[
            pl.BlockSpec((BN, K), lambda i: (i, 0)),
            pl.BlockSpec((BN, K), lambda i: (i, 0)),
            pl.BlockSpec((BN, 1), lambda i: (i, 0)),
            pl.BlockSpec((1, 1), lambda i: (0, 0)),
            pl.BlockSpec((1, 1), lambda i: (0, 0)),
        ],
        out_shape=[
            jax.ShapeDtypeStruct((N, K), jnp.float32),
            jax.ShapeDtypeStruct((N, K), jnp.float32),
            jax.ShapeDtypeStruct((N, 1), jnp.int32),
            jax.ShapeDtypeStruct((1, 1), jnp.float32),
            jax.ShapeDtypeStruct((1, 1), jnp.float32),
        ],
        scratch_shapes=[
            pltpu.VMEM((1, K), jnp.float32),
            pltpu.SMEM((1,), jnp.float32),
        ],
    )(x, e, x2, e2)


def _sc_body(emb_hbm, idx_hbm, x_hbm, out_hbm, idx_v, rows_v, x_v, sem):
    c = lax.axis_index("c")
    s = lax.axis_index("s")
    wid = s * 2 + c
    base = wid * _BPW
    pltpu.sync_copy(idx_hbm.at[pl.ds(wid * 2, 2), :], idx_v)
    cps = [pltpu.async_copy(emb_hbm.at[idx_v.at[j]],
                            rows_v.at[pl.ds(j * 128, 128), :], sem)
           for j in range(2)]
    pltpu.sync_copy(x_hbm.at[pl.ds(base, _BPW), :], x_v)
    for cp in cps:
        cp.wait()

    def body(r, carry):
        for ch in range(D // 16):
            sl = pl.ds(ch * 16, 16)
            u = lax.bitcast_convert_type(rows_v[r, sl], jnp.int32)
            u = u + 32767 + ((u >> 16) & 1)
            q = lax.bitcast_convert_type(u & jnp.int32(-65536), jnp.float32)
            xv = x_v[r, sl]
            rows_v[r, sl] = xv + (q - xv)
        return carry

    lax.fori_loop(0, _BPW, body, 0)
    pltpu.sync_copy(rows_v, out_hbm.at[pl.ds(base, _BPW), :])


def _sc_call(emb_bf, idx2d, x):
    mesh = plsc.VectorSubcoreMesh(core_axis_name="c", subcore_axis_name="s")
    fn = functools.partial(
        pl.kernel,
        mesh=mesh,
        out_type=jax.ShapeDtypeStruct((N, D), jnp.float32),
        scratch_types=[
            pltpu.VMEM((2, 128), jnp.int32),
            pltpu.VMEM((_BPW, D), jnp.float32),
            pltpu.VMEM((_BPW, D), jnp.float32),
            pltpu.SemaphoreType.DMA,
        ],
        compiler_params=pltpu.CompilerParams(use_tc_tiling_on_sc=False),
    )(_sc_body)
    return fn(emb_bf, idx2d, x)


def kernel(inputs, embedding):
    x = inputs.reshape(-1, D)
    x2 = jnp.sum(x ** 2, axis=1, keepdims=True)
    e2 = jnp.sum(embedding ** 2, axis=1)[None, :]
    dist, enc, idx, loss11, perp11 = _tc_call(x, embedding, x2, e2)
    qst = _sc_call(embedding, idx.reshape(N // 128, 128), x)
    return (loss11[0, 0], qst.reshape(inputs.shape), perp11[0, 0],
            enc, dist, idx)
